# baseline (speedup 1.0000x reference)
import functools

import jax
import jax.numpy as jnp
from jax import lax
from jax.experimental import pallas as pl
from jax.experimental.pallas import tpu as pltpu
from jax.experimental.pallas import tpu_sc as plsc

VOCAB = 1000000
EMBED_DIM = 64
N_TOKENS = 20
BATCH = 1024
SEQ = 200

NUM_CORES = 2
NUM_SUBCORES = 16
NUM_WORKERS = NUM_CORES * NUM_SUBCORES
ROWS_PER_WORKER = BATCH // NUM_WORKERS

_CHUNK0 = 128
_CHUNK1 = SEQ - _CHUNK0

_mesh = plsc.VectorSubcoreMesh(core_axis_name="c", subcore_axis_name="s")


@functools.partial(
    pl.kernel,
    mesh=_mesh,
    out_type=jax.ShapeDtypeStruct((BATCH, N_TOKENS + SEQ, EMBED_DIM), jnp.float32),
    scratch_types=[
        pltpu.VMEM((SEQ,), jnp.int32),
        pltpu.VMEM((N_TOKENS + SEQ, EMBED_DIM), jnp.float32),
        pltpu.SemaphoreType.DMA,
    ],
    compiler_params=pltpu.CompilerParams(use_tc_tiling_on_sc=False),
)
def _soft_embed(ids_hbm, table_hbm, lemb_hbm, out_hbm, idx_v, rows_v, sem):
    wid = lax.axis_index("s") * NUM_CORES + lax.axis_index("c")
    base = wid * ROWS_PER_WORKER
    pltpu.sync_copy(lemb_hbm, rows_v.at[pl.ds(0, N_TOKENS)])

    def body(i, carry):
        b = base + i
        pltpu.sync_copy(ids_hbm.at[b], idx_v)
        cp0 = pltpu.async_copy(
            table_hbm.at[idx_v.at[pl.ds(0, _CHUNK0)]],
            rows_v.at[pl.ds(N_TOKENS, _CHUNK0)],
            sem,
        )
        cp1 = pltpu.async_copy(
            table_hbm.at[idx_v.at[pl.ds(_CHUNK0, _CHUNK1)]],
            rows_v.at[pl.ds(N_TOKENS + _CHUNK0, _CHUNK1)],
            sem,
        )
        cp0.wait()
        cp1.wait()
        pltpu.sync_copy(rows_v, out_hbm.at[b])
        return carry

    lax.fori_loop(0, ROWS_PER_WORKER, body, 0)


def kernel(input_ids, wte_weight, learned_embedding):
    ids = input_ids.astype(jnp.int32)
    return _soft_embed(ids, wte_weight, learned_embedding)


# speedup vs baseline: 1.0443x; 1.0443x over previous
import functools

import jax
import jax.numpy as jnp
from jax import lax
from jax.experimental import pallas as pl
from jax.experimental.pallas import tpu as pltpu
from jax.experimental.pallas import tpu_sc as plsc

VOCAB = 1000000
EMBED_DIM = 64
N_TOKENS = 20
BATCH = 1024
SEQ = 200
OUT_ROWS = N_TOKENS + SEQ

NUM_CORES = 2
NUM_SUBCORES = 16
NUM_WORKERS = NUM_CORES * NUM_SUBCORES
ROWS_PER_WORKER = BATCH // NUM_WORKERS

_CHUNK0 = 128
_CHUNK1 = SEQ - _CHUNK0
_NBUF = 4
_T = ROWS_PER_WORKER // _NBUF

_mesh = plsc.VectorSubcoreMesh(core_axis_name="c", subcore_axis_name="s")


@functools.partial(
    pl.kernel,
    mesh=_mesh,
    out_type=jax.ShapeDtypeStruct((BATCH, OUT_ROWS, EMBED_DIM), jnp.float32),
    scratch_types=[
        pltpu.VMEM((ROWS_PER_WORKER, SEQ), jnp.int32),
        pltpu.VMEM((_NBUF, OUT_ROWS, EMBED_DIM), jnp.float32),
        pltpu.SemaphoreType.DMA((_NBUF,)),
        pltpu.SemaphoreType.DMA((_NBUF,)),
    ],
    compiler_params=pltpu.CompilerParams(use_tc_tiling_on_sc=False),
)
def _soft_embed(ids_hbm, table_hbm, lemb_hbm, out_hbm, idx_v, bufs, gsem, osem):
    wid = lax.axis_index("s") * NUM_CORES + lax.axis_index("c")
    base = wid * ROWS_PER_WORKER

    def start_gathers(i, b):
        pltpu.async_copy(
            table_hbm.at[idx_v.at[i, pl.ds(0, _CHUNK0)]],
            bufs.at[b, pl.ds(N_TOKENS, _CHUNK0)],
            gsem.at[b],
        )
        pltpu.async_copy(
            table_hbm.at[idx_v.at[i, pl.ds(_CHUNK0, _CHUNK1)]],
            bufs.at[b, pl.ds(N_TOKENS + _CHUNK0, _CHUNK1)],
            gsem.at[b],
        )

    def wait_gathers(i, b):
        pltpu.make_async_copy(
            table_hbm.at[idx_v.at[i, pl.ds(0, _CHUNK0)]],
            bufs.at[b, pl.ds(N_TOKENS, _CHUNK0)],
            gsem.at[b],
        ).wait()
        pltpu.make_async_copy(
            table_hbm.at[idx_v.at[i, pl.ds(_CHUNK0, _CHUNK1)]],
            bufs.at[b, pl.ds(N_TOKENS + _CHUNK0, _CHUNK1)],
            gsem.at[b],
        ).wait()

    def start_out(i, b):
        pltpu.async_copy(bufs.at[b], out_hbm.at[base + i], osem.at[b])

    def wait_out(i, b):
        pltpu.make_async_copy(bufs.at[b], out_hbm.at[base + i], osem.at[b]).wait()

    pltpu.sync_copy(ids_hbm.at[pl.ds(base, ROWS_PER_WORKER)], idx_v)
    for b in range(_NBUF):
        pltpu.sync_copy(lemb_hbm, bufs.at[b, pl.ds(0, N_TOKENS)])
        start_gathers(b, b)

    def body(t, carry):
        for b in range(_NBUF):
            i = t * _NBUF + b
            wait_gathers(i, b)
            start_out(i, b)
            wait_out(i, b)
            start_gathers(i + _NBUF, b)
        return carry

    lax.fori_loop(0, _T - 1, body, 0)

    for b in range(_NBUF):
        i = (_T - 1) * _NBUF + b
        wait_gathers(i, b)
        start_out(i, b)
    for b in range(_NBUF):
        i = (_T - 1) * _NBUF + b
        wait_out(i, b)


def kernel(input_ids, wte_weight, learned_embedding):
    ids = input_ids.astype(jnp.int32)
    return _soft_embed(ids, wte_weight, learned_embedding)


# speedup vs baseline: 1.1986x; 1.1477x over previous
import functools

import jax
import jax.numpy as jnp
from jax import lax
from jax.experimental import pallas as pl
from jax.experimental.pallas import tpu as pltpu
from jax.experimental.pallas import tpu_sc as plsc

VOCAB = 1000000
EMBED_DIM = 64
PAD_DIM = 128
N_TOKENS = 20
BATCH = 1024
SEQ = 200
OUT_ROWS = N_TOKENS + SEQ

NUM_CORES = 2
NUM_WORKERS = 32
ROWS_PER_WORKER = BATCH // NUM_WORKERS

_CHUNK0 = 128
_CHUNK1 = SEQ - _CHUNK0
_NBUF = 2

_mesh = plsc.VectorSubcoreMesh(core_axis_name="c", subcore_axis_name="s")


@functools.partial(
    pl.kernel,
    mesh=_mesh,
    out_type=jax.ShapeDtypeStruct((BATCH, OUT_ROWS, PAD_DIM), jnp.float32),
    scratch_types=[
        pltpu.VMEM((_NBUF, SEQ), jnp.int32),
        pltpu.VMEM((_NBUF, OUT_ROWS, PAD_DIM), jnp.float32),
        pltpu.SemaphoreType.DMA((_NBUF,)),
        pltpu.SemaphoreType.DMA((_NBUF,)),
    ],
)
def _soft_embed(ids_hbm, table_hbm, lemb_hbm, out_hbm, idx_v, bufs, gsem, osem):
    wid = lax.axis_index("s") * NUM_CORES + lax.axis_index("c")
    base = wid * ROWS_PER_WORKER

    def load_ids(i, b):
        pltpu.sync_copy(ids_hbm.at[base + i], idx_v.at[b])

    def start_gathers(b):
        pltpu.async_copy(
            table_hbm.at[idx_v.at[b, pl.ds(0, _CHUNK0)]],
            bufs.at[b, pl.ds(N_TOKENS, _CHUNK0)],
            gsem.at[b],
        )
        pltpu.async_copy(
            table_hbm.at[idx_v.at[b, pl.ds(_CHUNK0, _CHUNK1)]],
            bufs.at[b, pl.ds(N_TOKENS + _CHUNK0, _CHUNK1)],
            gsem.at[b],
        )

    def wait_gathers(b):
        pltpu.make_async_copy(
            table_hbm.at[idx_v.at[b, pl.ds(0, _CHUNK0)]],
            bufs.at[b, pl.ds(N_TOKENS, _CHUNK0)],
            gsem.at[b],
        ).wait()
        pltpu.make_async_copy(
            table_hbm.at[idx_v.at[b, pl.ds(_CHUNK0, _CHUNK1)]],
            bufs.at[b, pl.ds(N_TOKENS + _CHUNK0, _CHUNK1)],
            gsem.at[b],
        ).wait()

    def start_out(i, b):
        pltpu.async_copy(bufs.at[b], out_hbm.at[base + i], osem.at[b])

    def wait_out(i, b):
        pltpu.make_async_copy(bufs.at[b], out_hbm.at[base + i], osem.at[b]).wait()

    for b in range(_NBUF):
        pltpu.sync_copy(lemb_hbm, bufs.at[b, pl.ds(0, N_TOKENS)])
        load_ids(b, b)
        start_gathers(b)

    _T = ROWS_PER_WORKER // _NBUF

    def body(t, carry):
        for b in range(_NBUF):
            i = t * _NBUF + b
            wait_gathers(b)
            start_out(i, b)
            wait_out(i, b)
            load_ids(i + _NBUF, b)
            start_gathers(b)
        return carry

    lax.fori_loop(0, _T - 1, body, 0)

    for b in range(_NBUF):
        i = (_T - 1) * _NBUF + b
        wait_gathers(b)
        start_out(i, b)
    for b in range(_NBUF):
        i = (_T - 1) * _NBUF + b
        wait_out(i, b)


def kernel(input_ids, wte_weight, learned_embedding):
    ids = input_ids.astype(jnp.int32)
    table128 = jnp.pad(wte_weight, ((0, 0), (0, PAD_DIM - EMBED_DIM)))
    lemb128 = jnp.pad(learned_embedding, ((0, 0), (0, PAD_DIM - EMBED_DIM)))
    big = _soft_embed(ids, table128, lemb128)
    return big[:, :, :EMBED_DIM]
